# Initial kernel scaffold; baseline (speedup 1.0000x reference)
import jax
import jax.numpy as jnp
from jax.experimental import pallas as pl


def kernel(x, edge_index, W1l, b1, W1r, W2l, b2, W2r, W3l, b3, W3r, Wal, ba, War, Wsl, bs, Wsr, Wel, be, Wer):
    raise NotImplementedError("write your pallas kernel here")


# speedup vs baseline: 3.5101x; 3.5101x over previous
import functools

import jax
import jax.numpy as jnp
from jax import lax
from jax.experimental import pallas as pl
from jax.experimental.pallas import tpu as pltpu
from jax.experimental.pallas import tpu_sc as plsc

N_NODES = 10000
N_EDGES = 320000
D = 128

NC, NS = 2, 16
CH = 128
EPT = 10240
NCH = EPT // CH
E_PAD = NC * NS * EPT
ROWS_A = 632
ROWS_B = N_NODES - (NS - 1) * ROWS_A
N_ACC = N_NODES + 16


def _sc_agg_body(dv, *refs):
    (table, src4, dst4, zeros_d,
     out, src_v, dst_v, rows_v, sem, acc_sh) = refs
    c = lax.axis_index("c")
    s = lax.axis_index("s")

    def rows_copy(mk_src, mk_dst):
        @pl.when(s < NS - 1)
        def _():
            pltpu.sync_copy(mk_src(s * ROWS_A, ROWS_A), mk_dst(s * ROWS_A, ROWS_A))

        @pl.when(s == NS - 1)
        def _():
            pltpu.sync_copy(mk_src((NS - 1) * ROWS_A, ROWS_B),
                            mk_dst((NS - 1) * ROWS_A, ROWS_B))

    rows_copy(lambda o, n: zeros_d.at[pl.ds(o, n)],
              lambda o, n: acc_sh.at[pl.ds(o, n)])

    pltpu.sync_copy(src4.at[c, s], src_v)
    pltpu.sync_copy(dst4.at[c, s], dst_v)
    plsc.subcore_barrier()

    def chunk(j, carry):
        pltpu.async_copy(table.at[src_v.at[j]], rows_v, sem).wait()
        pltpu.sync_copy(rows_v, acc_sh.at[dst_v.at[j]], add=True)
        return carry

    lax.fori_loop(0, NCH, chunk, 0)
    plsc.subcore_barrier()

    rows_copy(lambda o, n: acc_sh.at[pl.ds(o, n)],
              lambda o, n: out.at[c, pl.ds(o, n)])


@functools.lru_cache(maxsize=None)
def _make_sc_agg(dv):
    mesh = plsc.VectorSubcoreMesh(core_axis_name="c", subcore_axis_name="s",
                                  num_cores=NC, num_subcores=NS)
    scratch = [
        pltpu.VMEM((NCH, CH), jnp.int32),
        pltpu.VMEM((NCH, CH), jnp.int32),
        pltpu.VMEM((CH, dv), jnp.float32),
        pltpu.SemaphoreType.DMA,
        pltpu.VMEM_SHARED((N_ACC, dv), jnp.float32),
    ]
    return pl.kernel(
        functools.partial(_sc_agg_body, dv),
        out_type=jax.ShapeDtypeStruct((NC, N_NODES, dv), jnp.float32),
        mesh=mesh,
        scratch_types=scratch,
        compiler_params=pltpu.CompilerParams(use_tc_tiling_on_sc=(dv == D)),
    )


def _dense_body(relu, proj, agg, cnt, h, wl, wr, b, *rest):
    c = cnt[0, :, 0:1] + cnt[1, :, 0:1]
    inv = 1.0 / jnp.maximum(c, 1.0)
    mean = (agg[0] + agg[1]) * inv
    y = (jnp.dot(mean, wl[...], preferred_element_type=jnp.float32)
         + b[...]
         + jnp.dot(h[...], wr[...], preferred_element_type=jnp.float32))
    if relu:
        y = jnp.maximum(y, 0.0)
    if proj:
        wp, out0, out1 = rest
        out0[...] = y
        out1[...] = jnp.dot(y, wp[...], preferred_element_type=jnp.float32)
    else:
        rest[0][...] = y


def _make_dense(relu, proj, dv):
    bm = 1000
    grid = N_NODES // bm
    in_specs = [
        pl.BlockSpec((NC, bm, dv), lambda i: (0, i, 0)),
        pl.BlockSpec((NC, bm, 16), lambda i: (0, i, 0)),
        pl.BlockSpec((bm, D), lambda i: (i, 0)),
        pl.BlockSpec((dv, dv), lambda i: (0, 0)),
        pl.BlockSpec((D, dv), lambda i: (0, 0)),
        pl.BlockSpec((1, dv), lambda i: (0, 0)),
    ]
    out_shape = [jax.ShapeDtypeStruct((N_NODES, dv), jnp.float32)]
    out_specs = [pl.BlockSpec((bm, dv), lambda i: (i, 0))]
    if proj:
        in_specs.append(pl.BlockSpec((D, 16), lambda i: (0, 0)))
        out_shape.append(jax.ShapeDtypeStruct((N_NODES, 16), jnp.float32))
        out_specs.append(pl.BlockSpec((bm, 16), lambda i: (i, 0)))

    def wrapped(agg, cnt, h, wlT, wrT, b, *rest):
        return pl.pallas_call(
            functools.partial(_dense_body, relu, proj),
            grid=(grid,),
            in_specs=in_specs,
            out_specs=out_specs if proj else out_specs[0],
            out_shape=out_shape if proj else out_shape[0],
        )(agg, cnt, h, wlT, wrT, b, *rest)

    return wrapped


def _sc_agg(*a):
    return _make_sc_agg(D)(*a)


def _sc_agg16(*a):
    return _make_sc_agg(16)(*a)


_dense_relu = _make_dense(True, False, D)
_dense_relu_proj = _make_dense(True, True, D)
_dense_head = _make_dense(False, False, 16)


def kernel(x, edge_index, W1l, b1, W1r, W2l, b2, W2r, W3l, b3, W3r,
           Wal, ba, War, Wsl, bs, Wsr, Wel, be, Wer):
    ei = edge_index.astype(jnp.int32)
    src, dst = ei[0], ei[1]
    pad = E_PAD - N_EDGES
    srcp = jnp.concatenate([src, jnp.zeros((pad,), jnp.int32)])
    dstp = jnp.concatenate([dst, jnp.full((pad,), N_NODES, jnp.int32)])
    src4 = srcp.reshape(NC, NS, NCH, CH)
    dst4 = dstp.reshape(NC, NS, NCH, CH)

    zeros_d = jnp.zeros((N_NODES, D), jnp.float32)
    zeros16 = jnp.zeros((N_NODES, 16), jnp.float32)
    ones_nodes = jnp.ones((N_NODES, 16), jnp.float32)

    cnt = _sc_agg16(ones_nodes, src4, dst4, zeros16)
    agg1 = _sc_agg(x, src4, dst4, zeros_d)
    h1 = _dense_relu(agg1, cnt, x, W1l.T, W1r.T, b1[None])
    agg2 = _sc_agg(h1, src4, dst4, zeros_d)
    h2 = _dense_relu(agg2, cnt, h1, W2l.T, W2r.T, b2[None])
    agg3 = _sc_agg(h2, src4, dst4, zeros_d)

    WcatT = jnp.pad(jnp.concatenate([Wal, Wsl, Wel], 0), ((0, 8), (0, 0))).T
    WrcatT = jnp.pad(jnp.concatenate([War, Wsr, Wer], 0), ((0, 8), (0, 0))).T
    bcat = jnp.pad(jnp.concatenate([ba, bs, be]), (0, 8))

    h3, p16 = _dense_relu_proj(agg3, cnt, h2, W3l.T, W3r.T, b3[None], WcatT)
    aggp = _sc_agg16(p16, src4, dst4, zeros16)
    outc = _dense_head(aggp, cnt, h3, jnp.eye(16, dtype=jnp.float32),
                       WrcatT, bcat[None])
    return outc[:, :3], outc[:, 3:5], outc[:, 5:8]


# speedup vs baseline: 3.9432x; 1.1234x over previous
import functools

import jax
import jax.numpy as jnp
from jax import lax
from jax.experimental import pallas as pl
from jax.experimental.pallas import tpu as pltpu
from jax.experimental.pallas import tpu_sc as plsc

N_NODES = 10000
N_EDGES = 320000
D = 128
DH = D // 2

NC, NS = 2, 16
CH = 128
E_PAD = 327680
NCH = E_PAD // NS // CH
ROWS_A = 632
ROWS_B = N_NODES - (NS - 1) * ROWS_A
N_ACC = N_NODES + 16
N_HALF = N_NODES // NC
NH_ACC = N_HALF + 16
HROWS_A = 312
HROWS_B = N_HALF - (NS - 1) * HROWS_A


def _pipeline(table, src_v, dst_v, rows2, sems, acc_sh, nch):
    sem_g = sems[:2]
    sem_s = sems[2:]

    def g_start(j, b):
        pltpu.async_copy(table.at[src_v.at[j]], rows2.at[b], sem_g[b])

    def g_wait(b):
        pltpu.make_async_copy(table.at[pl.ds(0, CH)], rows2.at[b],
                              sem_g[b]).wait()

    def s_start(j, b):
        pltpu.async_copy(rows2.at[b], acc_sh.at[dst_v.at[j]], sem_s[b],
                         add=True)

    def s_wait(b):
        pltpu.make_async_copy(rows2.at[b], acc_sh.at[pl.ds(0, CH)],
                              sem_s[b]).wait()

    def pair(p, first, last):
        g_wait(0)
        if not first:
            s_wait(1)
        s_start(2 * p, 0)
        g_start(2 * p + 1, 1)
        g_wait(1)
        s_wait(0)
        s_start(2 * p + 1, 1)
        if not last:
            g_start(2 * p + 2, 0)

    np_ = nch // 2

    def steady(p, carry):
        pair(p, False, False)
        return carry

    g_start(0, 0)
    pair(0, True, False)
    lax.fori_loop(1, np_ - 1, steady, 0)
    pair(np_ - 1, False, True)
    s_wait(1)


def _sc_agg_feat_body(*refs):
    (tabA, tabB, src3, dst3, zeros_h, outA, outB, src_v, dst_v,
     rows2, sem_g0, sem_g1, sem_s0, sem_s1, acc_sh) = refs
    c = lax.axis_index("c")
    s = lax.axis_index("s")

    def rows_copy(mk_src, mk_dst):
        @pl.when(s < NS - 1)
        def _():
            pltpu.sync_copy(mk_src(s * ROWS_A, ROWS_A),
                            mk_dst(s * ROWS_A, ROWS_A))

        @pl.when(s == NS - 1)
        def _():
            pltpu.sync_copy(mk_src((NS - 1) * ROWS_A, ROWS_B),
                            mk_dst((NS - 1) * ROWS_A, ROWS_B))

    rows_copy(lambda o, n: zeros_h.at[pl.ds(o, n)],
              lambda o, n: acc_sh.at[pl.ds(o, n)])
    pltpu.sync_copy(src3.at[s], src_v)
    pltpu.sync_copy(dst3.at[s], dst_v)
    plsc.subcore_barrier()

    sems = (sem_g0, sem_g1, sem_s0, sem_s1)

    @pl.when(c == 0)
    def _():
        _pipeline(tabA, src_v, dst_v, rows2, sems, acc_sh, NCH)

    @pl.when(c == 1)
    def _():
        _pipeline(tabB, src_v, dst_v, rows2, sems, acc_sh, NCH)

    plsc.subcore_barrier()

    @pl.when(c == 0)
    def _():
        rows_copy(lambda o, n: acc_sh.at[pl.ds(o, n)],
                  lambda o, n: outA.at[pl.ds(o, n)])

    @pl.when(c == 1)
    def _():
        rows_copy(lambda o, n: acc_sh.at[pl.ds(o, n)],
                  lambda o, n: outB.at[pl.ds(o, n)])


def _sc_agg_node16_body(*refs):
    (table, src3, dst4, zeros16, out, src_v, dst_v,
     rows2, sem_g0, sem_g1, sem_s0, sem_s1, acc_sh) = refs
    c = lax.axis_index("c")
    s = lax.axis_index("s")

    def rows_copy(mk_src, mk_dst):
        @pl.when(s < NS - 1)
        def _():
            pltpu.sync_copy(mk_src(s * HROWS_A, HROWS_A),
                            mk_dst(s * HROWS_A, HROWS_A))

        @pl.when(s == NS - 1)
        def _():
            pltpu.sync_copy(mk_src((NS - 1) * HROWS_A, HROWS_B),
                            mk_dst((NS - 1) * HROWS_A, HROWS_B))

    rows_copy(lambda o, n: zeros16.at[pl.ds(o, n)],
              lambda o, n: acc_sh.at[pl.ds(o, n)])
    pltpu.sync_copy(src3.at[s], src_v)
    pltpu.sync_copy(dst4.at[c, s], dst_v)
    plsc.subcore_barrier()

    _pipeline(table, src_v, dst_v, rows2,
              (sem_g0, sem_g1, sem_s0, sem_s1), acc_sh, NCH)
    plsc.subcore_barrier()

    rows_copy(lambda o, n: acc_sh.at[pl.ds(o, n)],
              lambda o, n: out.at[pl.ds(c * N_HALF + o, n)])


@functools.lru_cache(maxsize=None)
def _make_sc_agg_feat():
    mesh = plsc.VectorSubcoreMesh(core_axis_name="c", subcore_axis_name="s",
                                  num_cores=NC, num_subcores=NS)
    return pl.kernel(
        _sc_agg_feat_body,
        out_type=(jax.ShapeDtypeStruct((N_NODES, DH), jnp.float32),
                  jax.ShapeDtypeStruct((N_NODES, DH), jnp.float32)),
        mesh=mesh,
        scratch_types=[
            pltpu.VMEM((NCH, CH), jnp.int32),
            pltpu.VMEM((NCH, CH), jnp.int32),
            pltpu.VMEM((2, CH, DH), jnp.float32),
            pltpu.SemaphoreType.DMA,
            pltpu.SemaphoreType.DMA,
            pltpu.SemaphoreType.DMA,
            pltpu.SemaphoreType.DMA,
            pltpu.VMEM_SHARED((N_ACC, DH), jnp.float32),
        ],
        compiler_params=pltpu.CompilerParams(use_tc_tiling_on_sc=False),
    )


@functools.lru_cache(maxsize=None)
def _make_sc_agg_node16():
    mesh = plsc.VectorSubcoreMesh(core_axis_name="c", subcore_axis_name="s",
                                  num_cores=NC, num_subcores=NS)
    return pl.kernel(
        _sc_agg_node16_body,
        out_type=jax.ShapeDtypeStruct((N_NODES, 16), jnp.float32),
        mesh=mesh,
        scratch_types=[
            pltpu.VMEM((NCH, CH), jnp.int32),
            pltpu.VMEM((NCH, CH), jnp.int32),
            pltpu.VMEM((2, CH, 16), jnp.float32),
            pltpu.SemaphoreType.DMA,
            pltpu.SemaphoreType.DMA,
            pltpu.SemaphoreType.DMA,
            pltpu.SemaphoreType.DMA,
            pltpu.VMEM_SHARED((NH_ACC, 16), jnp.float32),
        ],
        compiler_params=pltpu.CompilerParams(use_tc_tiling_on_sc=False),
    )


def _dense_body(relu, proj, aggA, aggB, cnt, hA, hB, wl, wr, b, *rest):
    inv = 1.0 / jnp.maximum(cnt[:, 0:1], 1.0)
    y = (jnp.dot(aggA[...] * inv, wl[0:DH], preferred_element_type=jnp.float32)
         + jnp.dot(aggB[...] * inv, wl[DH:D],
                   preferred_element_type=jnp.float32)
         + b[...]
         + jnp.dot(hA[...], wr[0:DH], preferred_element_type=jnp.float32)
         + jnp.dot(hB[...], wr[DH:D], preferred_element_type=jnp.float32))
    if relu:
        y = jnp.maximum(y, 0.0)
    if proj:
        wp, oA, oB, op16 = rest
        oA[...] = y[:, 0:DH]
        oB[...] = y[:, DH:D]
        op16[...] = jnp.dot(y, wp[...], preferred_element_type=jnp.float32)
    else:
        oA, oB = rest
        oA[...] = y[:, 0:DH]
        oB[...] = y[:, DH:D]


BM = 1000


def _make_dense(relu, proj):
    grid = N_NODES // BM
    in_specs = [
        pl.BlockSpec((BM, DH), lambda i: (i, 0)),
        pl.BlockSpec((BM, DH), lambda i: (i, 0)),
        pl.BlockSpec((BM, 16), lambda i: (i, 0)),
        pl.BlockSpec((BM, DH), lambda i: (i, 0)),
        pl.BlockSpec((BM, DH), lambda i: (i, 0)),
        pl.BlockSpec((D, D), lambda i: (0, 0)),
        pl.BlockSpec((D, D), lambda i: (0, 0)),
        pl.BlockSpec((1, D), lambda i: (0, 0)),
    ]
    out_shape = [jax.ShapeDtypeStruct((N_NODES, DH), jnp.float32),
                 jax.ShapeDtypeStruct((N_NODES, DH), jnp.float32)]
    out_specs = [pl.BlockSpec((BM, DH), lambda i: (i, 0)),
                 pl.BlockSpec((BM, DH), lambda i: (i, 0))]
    if proj:
        in_specs.append(pl.BlockSpec((D, 16), lambda i: (0, 0)))
        out_shape.append(jax.ShapeDtypeStruct((N_NODES, 16), jnp.float32))
        out_specs.append(pl.BlockSpec((BM, 16), lambda i: (i, 0)))

    def wrapped(*args):
        return pl.pallas_call(
            functools.partial(_dense_body, relu, proj),
            grid=(grid,),
            in_specs=in_specs,
            out_specs=out_specs,
            out_shape=out_shape,
        )(*args)

    return wrapped


def _head_body(aggp, cnt, hA, hB, wr, b, out):
    inv = 1.0 / jnp.maximum(cnt[:, 0:1], 1.0)
    out[...] = (aggp[...] * inv + b[...]
                + jnp.dot(hA[...], wr[0:DH],
                          preferred_element_type=jnp.float32)
                + jnp.dot(hB[...], wr[DH:D],
                          preferred_element_type=jnp.float32))


def _head_dense(aggp, cnt, hA, hB, wrcatT, bcat):
    grid = N_NODES // BM
    return pl.pallas_call(
        _head_body,
        grid=(grid,),
        in_specs=[
            pl.BlockSpec((BM, 16), lambda i: (i, 0)),
            pl.BlockSpec((BM, 16), lambda i: (i, 0)),
            pl.BlockSpec((BM, DH), lambda i: (i, 0)),
            pl.BlockSpec((BM, DH), lambda i: (i, 0)),
            pl.BlockSpec((D, 16), lambda i: (0, 0)),
            pl.BlockSpec((1, 16), lambda i: (0, 0)),
        ],
        out_specs=pl.BlockSpec((BM, 16), lambda i: (i, 0)),
        out_shape=jax.ShapeDtypeStruct((N_NODES, 16), jnp.float32),
    )(aggp, cnt, hA, hB, wrcatT, bcat)


def kernel(x, edge_index, W1l, b1, W1r, W2l, b2, W2r, W3l, b3, W3r,
           Wal, ba, War, Wsl, bs, Wsr, Wel, be, Wer):
    ei = edge_index.astype(jnp.int32)
    src, dst = ei[0], ei[1]
    pad = E_PAD - N_EDGES
    srcp = jnp.concatenate([src, jnp.zeros((pad,), jnp.int32)])
    dstp = jnp.concatenate([dst, jnp.full((pad,), N_NODES, jnp.int32)])
    src3 = srcp.reshape(NS, NCH, CH)
    dst3 = dstp.reshape(NS, NCH, CH)
    half = dstp // N_HALF
    local = dstp - half * N_HALF
    dst4h = jnp.stack([
        jnp.where(half == c2, local, N_HALF) for c2 in range(NC)
    ]).reshape(NC, NS, NCH, CH)

    zeros_h = jnp.zeros((N_NODES, DH), jnp.float32)
    zeros16 = jnp.zeros((N_NODES, 16), jnp.float32)
    ones16 = jnp.ones((N_NODES, 16), jnp.float32)

    agg_feat = _make_sc_agg_feat()
    agg_n16 = _make_sc_agg_node16()
    dense_relu = _make_dense(True, False)
    dense_relu_proj = _make_dense(True, True)

    cnt = agg_n16(ones16, src3, dst4h, zeros16)

    xA, xB = x[:, 0:DH], x[:, DH:D]
    a1A, a1B = agg_feat(xA, xB, src3, dst3, zeros_h)
    h1A, h1B = dense_relu(a1A, a1B, cnt, xA, xB, W1l.T, W1r.T, b1[None])
    a2A, a2B = agg_feat(h1A, h1B, src3, dst3, zeros_h)
    h2A, h2B = dense_relu(a2A, a2B, cnt, h1A, h1B, W2l.T, W2r.T, b2[None])
    a3A, a3B = agg_feat(h2A, h2B, src3, dst3, zeros_h)

    WcatT = jnp.pad(jnp.concatenate([Wal, Wsl, Wel], 0), ((0, 8), (0, 0))).T
    WrcatT = jnp.pad(jnp.concatenate([War, Wsr, Wer], 0), ((0, 8), (0, 0))).T
    bcat = jnp.pad(jnp.concatenate([ba, bs, be]), (0, 8))

    h3A, h3B, p16 = dense_relu_proj(a3A, a3B, cnt, h2A, h2B,
                                    W3l.T, W3r.T, b3[None], WcatT)
    aggp = agg_n16(p16, src3, dst4h, zeros16)
    outc = _head_dense(aggp, cnt, h3A, h3B, WrcatT, bcat[None])
    return outc[:, :3], outc[:, 3:5], outc[:, 5:8]


# speedup vs baseline: 4.2262x; 1.0718x over previous
import functools

import jax
import jax.numpy as jnp
from jax import lax
from jax.experimental import pallas as pl
from jax.experimental.pallas import tpu as pltpu
from jax.experimental.pallas import tpu_sc as plsc

N_NODES = 10000
N_EDGES = 320000
D = 128
DH = D // 2

NC, NS = 2, 16
E_PAD = 327680
CH_F = 256
CH_N = 512
NCH_F = E_PAD // NS // CH_F
NCH_N = E_PAD // NS // CH_N
ROWS_A = 632
ROWS_B = N_NODES - (NS - 1) * ROWS_A
N_ACC = N_NODES + 16
N_HALF = N_NODES // NC
NH_ACC = N_HALF + 16
HROWS_A = 312
HROWS_B = N_HALF - (NS - 1) * HROWS_A


def _pipeline(table, src_v, dst_v, rows2, sems, acc_sh, nch, ch):
    sem_g = sems[:2]
    sem_s = sems[2:]

    def g_start(j, b):
        pltpu.async_copy(table.at[src_v.at[j]], rows2.at[b], sem_g[b])

    def g_wait(b):
        pltpu.make_async_copy(table.at[pl.ds(0, ch)], rows2.at[b],
                              sem_g[b]).wait()

    def s_start(j, b):
        pltpu.async_copy(rows2.at[b], acc_sh.at[dst_v.at[j]], sem_s[b],
                         add=True)

    def s_wait(b):
        pltpu.make_async_copy(rows2.at[b], acc_sh.at[pl.ds(0, ch)],
                              sem_s[b]).wait()

    def pair(p, first, last):
        g_wait(0)
        if not first:
            s_wait(1)
        s_start(2 * p, 0)
        g_start(2 * p + 1, 1)
        g_wait(1)
        s_wait(0)
        s_start(2 * p + 1, 1)
        if not last:
            g_start(2 * p + 2, 0)

    np_ = nch // 2

    def steady(p, carry):
        pair(p, False, False)
        return carry

    g_start(0, 0)
    pair(0, True, False)
    lax.fori_loop(1, np_ - 1, steady, 0)
    pair(np_ - 1, False, True)
    s_wait(1)


def _sc_agg_feat_body(*refs):
    (tabA, tabB, src3, dst3, zeros_h, outA, outB, src_v, dst_v,
     rows2, sem_g0, sem_g1, sem_s0, sem_s1, acc_sh) = refs
    c = lax.axis_index("c")
    s = lax.axis_index("s")

    def rows_copy(mk_src, mk_dst):
        @pl.when(s < NS - 1)
        def _():
            pltpu.sync_copy(mk_src(s * ROWS_A, ROWS_A),
                            mk_dst(s * ROWS_A, ROWS_A))

        @pl.when(s == NS - 1)
        def _():
            pltpu.sync_copy(mk_src((NS - 1) * ROWS_A, ROWS_B),
                            mk_dst((NS - 1) * ROWS_A, ROWS_B))

    rows_copy(lambda o, n: zeros_h.at[pl.ds(o, n)],
              lambda o, n: acc_sh.at[pl.ds(o, n)])
    pltpu.sync_copy(src3.at[s], src_v)
    pltpu.sync_copy(dst3.at[s], dst_v)
    plsc.subcore_barrier()

    sems = (sem_g0, sem_g1, sem_s0, sem_s1)

    @pl.when(c == 0)
    def _():
        _pipeline(tabA, src_v, dst_v, rows2, sems, acc_sh, NCH_F, CH_F)

    @pl.when(c == 1)
    def _():
        _pipeline(tabB, src_v, dst_v, rows2, sems, acc_sh, NCH_F, CH_F)

    plsc.subcore_barrier()

    @pl.when(c == 0)
    def _():
        rows_copy(lambda o, n: acc_sh.at[pl.ds(o, n)],
                  lambda o, n: outA.at[pl.ds(o, n)])

    @pl.when(c == 1)
    def _():
        rows_copy(lambda o, n: acc_sh.at[pl.ds(o, n)],
                  lambda o, n: outB.at[pl.ds(o, n)])


def _sc_agg_node16_body(*refs):
    (table, src3, dst4, zeros16, out, src_v, dst_v,
     rows2, sem_g0, sem_g1, sem_s0, sem_s1, acc_sh) = refs
    c = lax.axis_index("c")
    s = lax.axis_index("s")

    def rows_copy(mk_src, mk_dst):
        @pl.when(s < NS - 1)
        def _():
            pltpu.sync_copy(mk_src(s * HROWS_A, HROWS_A),
                            mk_dst(s * HROWS_A, HROWS_A))

        @pl.when(s == NS - 1)
        def _():
            pltpu.sync_copy(mk_src((NS - 1) * HROWS_A, HROWS_B),
                            mk_dst((NS - 1) * HROWS_A, HROWS_B))

    rows_copy(lambda o, n: zeros16.at[pl.ds(o, n)],
              lambda o, n: acc_sh.at[pl.ds(o, n)])
    pltpu.sync_copy(src3.at[s], src_v)
    pltpu.sync_copy(dst4.at[c, s], dst_v)
    plsc.subcore_barrier()

    _pipeline(table, src_v, dst_v, rows2,
              (sem_g0, sem_g1, sem_s0, sem_s1), acc_sh, NCH_N, CH_N)
    plsc.subcore_barrier()

    rows_copy(lambda o, n: acc_sh.at[pl.ds(o, n)],
              lambda o, n: out.at[pl.ds(c * N_HALF + o, n)])


@functools.lru_cache(maxsize=None)
def _make_sc_agg_feat():
    mesh = plsc.VectorSubcoreMesh(core_axis_name="c", subcore_axis_name="s",
                                  num_cores=NC, num_subcores=NS)
    return pl.kernel(
        _sc_agg_feat_body,
        out_type=(jax.ShapeDtypeStruct((N_NODES, DH), jnp.float32),
                  jax.ShapeDtypeStruct((N_NODES, DH), jnp.float32)),
        mesh=mesh,
        scratch_types=[
            pltpu.VMEM((NCH_F, CH_F), jnp.int32),
            pltpu.VMEM((NCH_F, CH_F), jnp.int32),
            pltpu.VMEM((2, CH_F, DH), jnp.float32),
            pltpu.SemaphoreType.DMA,
            pltpu.SemaphoreType.DMA,
            pltpu.SemaphoreType.DMA,
            pltpu.SemaphoreType.DMA,
            pltpu.VMEM_SHARED((N_ACC, DH), jnp.float32),
        ],
        compiler_params=pltpu.CompilerParams(use_tc_tiling_on_sc=False),
    )


@functools.lru_cache(maxsize=None)
def _make_sc_agg_node16():
    mesh = plsc.VectorSubcoreMesh(core_axis_name="c", subcore_axis_name="s",
                                  num_cores=NC, num_subcores=NS)
    return pl.kernel(
        _sc_agg_node16_body,
        out_type=jax.ShapeDtypeStruct((N_NODES, 16), jnp.float32),
        mesh=mesh,
        scratch_types=[
            pltpu.VMEM((NCH_N, CH_N), jnp.int32),
            pltpu.VMEM((NCH_N, CH_N), jnp.int32),
            pltpu.VMEM((2, CH_N, 16), jnp.float32),
            pltpu.SemaphoreType.DMA,
            pltpu.SemaphoreType.DMA,
            pltpu.SemaphoreType.DMA,
            pltpu.SemaphoreType.DMA,
            pltpu.VMEM_SHARED((NH_ACC, 16), jnp.float32),
        ],
        compiler_params=pltpu.CompilerParams(use_tc_tiling_on_sc=False),
    )


def _dense_body(relu, proj, aggA, aggB, cnt, hA, hB, wl, wr, b, *rest):
    inv = 1.0 / jnp.maximum(cnt[:, 0:1], 1.0)
    y = (jnp.dot(aggA[...] * inv, wl[0:DH], preferred_element_type=jnp.float32)
         + jnp.dot(aggB[...] * inv, wl[DH:D],
                   preferred_element_type=jnp.float32)
         + b[...]
         + jnp.dot(hA[...], wr[0:DH], preferred_element_type=jnp.float32)
         + jnp.dot(hB[...], wr[DH:D], preferred_element_type=jnp.float32))
    if relu:
        y = jnp.maximum(y, 0.0)
    if proj:
        wp, oA, oB, op16 = rest
        oA[...] = y[:, 0:DH]
        oB[...] = y[:, DH:D]
        op16[...] = jnp.dot(y, wp[...], preferred_element_type=jnp.float32)
    else:
        oA, oB = rest
        oA[...] = y[:, 0:DH]
        oB[...] = y[:, DH:D]


BM = 1000


def _make_dense(relu, proj):
    grid = N_NODES // BM
    in_specs = [
        pl.BlockSpec((BM, DH), lambda i: (i, 0)),
        pl.BlockSpec((BM, DH), lambda i: (i, 0)),
        pl.BlockSpec((BM, 16), lambda i: (i, 0)),
        pl.BlockSpec((BM, DH), lambda i: (i, 0)),
        pl.BlockSpec((BM, DH), lambda i: (i, 0)),
        pl.BlockSpec((D, D), lambda i: (0, 0)),
        pl.BlockSpec((D, D), lambda i: (0, 0)),
        pl.BlockSpec((1, D), lambda i: (0, 0)),
    ]
    out_shape = [jax.ShapeDtypeStruct((N_NODES, DH), jnp.float32),
                 jax.ShapeDtypeStruct((N_NODES, DH), jnp.float32)]
    out_specs = [pl.BlockSpec((BM, DH), lambda i: (i, 0)),
                 pl.BlockSpec((BM, DH), lambda i: (i, 0))]
    if proj:
        in_specs.append(pl.BlockSpec((D, 16), lambda i: (0, 0)))
        out_shape.append(jax.ShapeDtypeStruct((N_NODES, 16), jnp.float32))
        out_specs.append(pl.BlockSpec((BM, 16), lambda i: (i, 0)))

    def wrapped(*args):
        return pl.pallas_call(
            functools.partial(_dense_body, relu, proj),
            grid=(grid,),
            in_specs=in_specs,
            out_specs=out_specs,
            out_shape=out_shape,
        )(*args)

    return wrapped


def _head_body(aggp, cnt, hA, hB, wr, b, out):
    inv = 1.0 / jnp.maximum(cnt[:, 0:1], 1.0)
    out[...] = (aggp[...] * inv + b[...]
                + jnp.dot(hA[...], wr[0:DH],
                          preferred_element_type=jnp.float32)
                + jnp.dot(hB[...], wr[DH:D],
                          preferred_element_type=jnp.float32))


def _head_dense(aggp, cnt, hA, hB, wrcatT, bcat):
    grid = N_NODES // BM
    return pl.pallas_call(
        _head_body,
        grid=(grid,),
        in_specs=[
            pl.BlockSpec((BM, 16), lambda i: (i, 0)),
            pl.BlockSpec((BM, 16), lambda i: (i, 0)),
            pl.BlockSpec((BM, DH), lambda i: (i, 0)),
            pl.BlockSpec((BM, DH), lambda i: (i, 0)),
            pl.BlockSpec((D, 16), lambda i: (0, 0)),
            pl.BlockSpec((1, 16), lambda i: (0, 0)),
        ],
        out_specs=pl.BlockSpec((BM, 16), lambda i: (i, 0)),
        out_shape=jax.ShapeDtypeStruct((N_NODES, 16), jnp.float32),
    )(aggp, cnt, hA, hB, wrcatT, bcat)


def kernel(x, edge_index, W1l, b1, W1r, W2l, b2, W2r, W3l, b3, W3r,
           Wal, ba, War, Wsl, bs, Wsr, Wel, be, Wer):
    ei = edge_index.astype(jnp.int32)
    src, dst = ei[0], ei[1]
    pad = E_PAD - N_EDGES
    srcp = jnp.concatenate([src, jnp.zeros((pad,), jnp.int32)])
    dstp = jnp.concatenate([dst, jnp.full((pad,), N_NODES, jnp.int32)])
    src3f = srcp.reshape(NS, NCH_F, CH_F)
    dst3f = dstp.reshape(NS, NCH_F, CH_F)
    src3n = srcp.reshape(NS, NCH_N, CH_N)
    half = dstp // N_HALF
    local = dstp - half * N_HALF
    dst4h = jnp.stack([
        jnp.where(half == c2, local, N_HALF) for c2 in range(NC)
    ]).reshape(NC, NS, NCH_N, CH_N)

    zeros_h = jnp.zeros((N_NODES, DH), jnp.float32)
    zeros16 = jnp.zeros((N_NODES, 16), jnp.float32)
    ones16 = jnp.ones((N_NODES, 16), jnp.float32)

    agg_feat = _make_sc_agg_feat()
    agg_n16 = _make_sc_agg_node16()
    dense_relu = _make_dense(True, False)
    dense_relu_proj = _make_dense(True, True)

    cnt = agg_n16(ones16, src3n, dst4h, zeros16)

    xA, xB = x[:, 0:DH], x[:, DH:D]
    a1A, a1B = agg_feat(xA, xB, src3f, dst3f, zeros_h)
    h1A, h1B = dense_relu(a1A, a1B, cnt, xA, xB, W1l.T, W1r.T, b1[None])
    a2A, a2B = agg_feat(h1A, h1B, src3f, dst3f, zeros_h)
    h2A, h2B = dense_relu(a2A, a2B, cnt, h1A, h1B, W2l.T, W2r.T, b2[None])
    a3A, a3B = agg_feat(h2A, h2B, src3f, dst3f, zeros_h)

    WcatT = jnp.pad(jnp.concatenate([Wal, Wsl, Wel], 0), ((0, 8), (0, 0))).T
    WrcatT = jnp.pad(jnp.concatenate([War, Wsr, Wer], 0), ((0, 8), (0, 0))).T
    bcat = jnp.pad(jnp.concatenate([ba, bs, be]), (0, 8))

    h3A, h3B, p16 = dense_relu_proj(a3A, a3B, cnt, h2A, h2B,
                                    W3l.T, W3r.T, b3[None], WcatT)
    aggp = agg_n16(p16, src3n, dst4h, zeros16)
    outc = _head_dense(aggp, cnt, h3A, h3B, WrcatT, bcat[None])
    return outc[:, :3], outc[:, 3:5], outc[:, 5:8]


# speedup vs baseline: 5.6328x; 1.3328x over previous
import functools

import jax
import jax.numpy as jnp
from jax import lax
from jax.experimental import pallas as pl
from jax.experimental.pallas import tpu as pltpu
from jax.experimental.pallas import tpu_sc as plsc

N_NODES = 10000
N_EDGES = 320000
D = 128
DH = D // 2

NC, NS = 2, 16
E_PAD = 327680
CH_F = 256
CH_N = 512
NCH_F = E_PAD // NS // CH_F
NCH_N = E_PAD // (NC * NS) // CH_N
ROWS_A = 632
ROWS_B = N_NODES - (NS - 1) * ROWS_A
N_ACC = N_NODES + 16
N_HALF = N_NODES // NC
NH_ACC = N_HALF + 16
HROWS_A = 312
HROWS_B = N_HALF - (NS - 1) * HROWS_A


def _pipeline(table, src_v, dst_v, rows2, sems, acc_sh, nch, ch):
    sem_g = sems[:2]
    sem_s = sems[2:]

    def g_start(j, b):
        pltpu.async_copy(table.at[src_v.at[j]], rows2.at[b], sem_g[b])

    def g_wait(b):
        pltpu.make_async_copy(table.at[pl.ds(0, ch)], rows2.at[b],
                              sem_g[b]).wait()

    def s_start(j, b):
        pltpu.async_copy(rows2.at[b], acc_sh.at[dst_v.at[j]], sem_s[b],
                         add=True)

    def s_wait(b):
        pltpu.make_async_copy(rows2.at[b], acc_sh.at[pl.ds(0, ch)],
                              sem_s[b]).wait()

    def pair(p, first, last):
        g_wait(0)
        if not first:
            s_wait(1)
        s_start(2 * p, 0)
        g_start(2 * p + 1, 1)
        g_wait(1)
        s_wait(0)
        s_start(2 * p + 1, 1)
        if not last:
            g_start(2 * p + 2, 0)

    np_ = nch // 2

    def steady(p, carry):
        pair(p, False, False)
        return carry

    g_start(0, 0)
    pair(0, True, False)
    lax.fori_loop(1, np_ - 1, steady, 0)
    pair(np_ - 1, False, True)
    s_wait(1)


def _sc_agg_feat_body(*refs):
    (tabA, tabB, src3, dst3, zeros_h, outA, outB, src_v, dst_v,
     rows2, sem_g0, sem_g1, sem_s0, sem_s1, acc_sh) = refs
    c = lax.axis_index("c")
    s = lax.axis_index("s")

    def rows_copy(mk_src, mk_dst):
        @pl.when(s < NS - 1)
        def _():
            pltpu.sync_copy(mk_src(s * ROWS_A, ROWS_A),
                            mk_dst(s * ROWS_A, ROWS_A))

        @pl.when(s == NS - 1)
        def _():
            pltpu.sync_copy(mk_src((NS - 1) * ROWS_A, ROWS_B),
                            mk_dst((NS - 1) * ROWS_A, ROWS_B))

    rows_copy(lambda o, n: zeros_h.at[pl.ds(o, n)],
              lambda o, n: acc_sh.at[pl.ds(o, n)])
    pltpu.sync_copy(src3.at[s], src_v)
    pltpu.sync_copy(dst3.at[s], dst_v)
    plsc.subcore_barrier()

    sems = (sem_g0, sem_g1, sem_s0, sem_s1)

    @pl.when(c == 0)
    def _():
        _pipeline(tabA, src_v, dst_v, rows2, sems, acc_sh, NCH_F, CH_F)

    @pl.when(c == 1)
    def _():
        _pipeline(tabB, src_v, dst_v, rows2, sems, acc_sh, NCH_F, CH_F)

    plsc.subcore_barrier()

    @pl.when(c == 0)
    def _():
        rows_copy(lambda o, n: acc_sh.at[pl.ds(o, n)],
                  lambda o, n: outA.at[pl.ds(o, n)])

    @pl.when(c == 1)
    def _():
        rows_copy(lambda o, n: acc_sh.at[pl.ds(o, n)],
                  lambda o, n: outB.at[pl.ds(o, n)])


def _sc_agg_node16_body(*refs):
    (table, src4, dst4, zeros16, out, src_v, dst_v,
     rows2, sem_g0, sem_g1, sem_s0, sem_s1, acc_sh) = refs
    c = lax.axis_index("c")
    s = lax.axis_index("s")

    def rows_copy(mk_src, mk_dst):
        @pl.when(s < NS - 1)
        def _():
            pltpu.sync_copy(mk_src(s * ROWS_A, ROWS_A),
                            mk_dst(s * ROWS_A, ROWS_A))

        @pl.when(s == NS - 1)
        def _():
            pltpu.sync_copy(mk_src((NS - 1) * ROWS_A, ROWS_B),
                            mk_dst((NS - 1) * ROWS_A, ROWS_B))

    rows_copy(lambda o, n: zeros16.at[pl.ds(o, n)],
              lambda o, n: acc_sh.at[pl.ds(o, n)])
    pltpu.sync_copy(src4.at[c, s], src_v)
    pltpu.sync_copy(dst4.at[c, s], dst_v)
    plsc.subcore_barrier()

    _pipeline(table, src_v, dst_v, rows2,
              (sem_g0, sem_g1, sem_s0, sem_s1), acc_sh, NCH_N, CH_N)
    plsc.subcore_barrier()

    rows_copy(lambda o, n: acc_sh.at[pl.ds(o, n)],
              lambda o, n: out.at[c, pl.ds(o, n)])


@functools.lru_cache(maxsize=None)
def _make_sc_agg_feat():
    mesh = plsc.VectorSubcoreMesh(core_axis_name="c", subcore_axis_name="s",
                                  num_cores=NC, num_subcores=NS)
    return pl.kernel(
        _sc_agg_feat_body,
        out_type=(jax.ShapeDtypeStruct((N_NODES, DH), jnp.float32),
                  jax.ShapeDtypeStruct((N_NODES, DH), jnp.float32)),
        mesh=mesh,
        scratch_types=[
            pltpu.VMEM((NCH_F, CH_F), jnp.int32),
            pltpu.VMEM((NCH_F, CH_F), jnp.int32),
            pltpu.VMEM((2, CH_F, DH), jnp.float32),
            pltpu.SemaphoreType.DMA,
            pltpu.SemaphoreType.DMA,
            pltpu.SemaphoreType.DMA,
            pltpu.SemaphoreType.DMA,
            pltpu.VMEM_SHARED((N_ACC, DH), jnp.float32),
        ],
        compiler_params=pltpu.CompilerParams(use_tc_tiling_on_sc=False),
    )


@functools.lru_cache(maxsize=None)
def _make_sc_agg_node16():
    mesh = plsc.VectorSubcoreMesh(core_axis_name="c", subcore_axis_name="s",
                                  num_cores=NC, num_subcores=NS)
    return pl.kernel(
        _sc_agg_node16_body,
        out_type=jax.ShapeDtypeStruct((NC, N_NODES, 16), jnp.float32),
        mesh=mesh,
        scratch_types=[
            pltpu.VMEM((NCH_N, CH_N), jnp.int32),
            pltpu.VMEM((NCH_N, CH_N), jnp.int32),
            pltpu.VMEM((2, CH_N, 16), jnp.float32),
            pltpu.SemaphoreType.DMA,
            pltpu.SemaphoreType.DMA,
            pltpu.SemaphoreType.DMA,
            pltpu.SemaphoreType.DMA,
            pltpu.VMEM_SHARED((N_ACC, 16), jnp.float32),
        ],
        compiler_params=pltpu.CompilerParams(use_tc_tiling_on_sc=False),
    )


def _dense_body(relu, proj, aggA, aggB, cnt, hA, hB, wl, wr, b, *rest):
    inv = 1.0 / jnp.maximum(cnt[0, :, 0:1] + cnt[1, :, 0:1], 1.0)
    y = (jnp.dot(aggA[...] * inv, wl[0:DH], preferred_element_type=jnp.float32)
         + jnp.dot(aggB[...] * inv, wl[DH:D],
                   preferred_element_type=jnp.float32)
         + b[...]
         + jnp.dot(hA[...], wr[0:DH], preferred_element_type=jnp.float32)
         + jnp.dot(hB[...], wr[DH:D], preferred_element_type=jnp.float32))
    if relu:
        y = jnp.maximum(y, 0.0)
    if proj:
        wp, oA, oB, op16 = rest
        oA[...] = y[:, 0:DH]
        oB[...] = y[:, DH:D]
        op16[...] = jnp.dot(y, wp[...], preferred_element_type=jnp.float32)
    else:
        oA, oB = rest
        oA[...] = y[:, 0:DH]
        oB[...] = y[:, DH:D]


BM = 1000


def _make_dense(relu, proj):
    grid = N_NODES // BM
    in_specs = [
        pl.BlockSpec((BM, DH), lambda i: (i, 0)),
        pl.BlockSpec((BM, DH), lambda i: (i, 0)),
        pl.BlockSpec((NC, BM, 16), lambda i: (0, i, 0)),
        pl.BlockSpec((BM, DH), lambda i: (i, 0)),
        pl.BlockSpec((BM, DH), lambda i: (i, 0)),
        pl.BlockSpec((D, D), lambda i: (0, 0)),
        pl.BlockSpec((D, D), lambda i: (0, 0)),
        pl.BlockSpec((1, D), lambda i: (0, 0)),
    ]
    out_shape = [jax.ShapeDtypeStruct((N_NODES, DH), jnp.float32),
                 jax.ShapeDtypeStruct((N_NODES, DH), jnp.float32)]
    out_specs = [pl.BlockSpec((BM, DH), lambda i: (i, 0)),
                 pl.BlockSpec((BM, DH), lambda i: (i, 0))]
    if proj:
        in_specs.append(pl.BlockSpec((D, 16), lambda i: (0, 0)))
        out_shape.append(jax.ShapeDtypeStruct((N_NODES, 16), jnp.float32))
        out_specs.append(pl.BlockSpec((BM, 16), lambda i: (i, 0)))

    def wrapped(*args):
        return pl.pallas_call(
            functools.partial(_dense_body, relu, proj),
            grid=(grid,),
            in_specs=in_specs,
            out_specs=out_specs,
            out_shape=out_shape,
        )(*args)

    return wrapped


def _head_body(aggp, cnt, hA, hB, wr, b, out):
    inv = 1.0 / jnp.maximum(cnt[0, :, 0:1] + cnt[1, :, 0:1], 1.0)
    out[...] = ((aggp[0] + aggp[1]) * inv + b[...]
                + jnp.dot(hA[...], wr[0:DH],
                          preferred_element_type=jnp.float32)
                + jnp.dot(hB[...], wr[DH:D],
                          preferred_element_type=jnp.float32))


def _head_dense(aggp, cnt, hA, hB, wrcatT, bcat):
    grid = N_NODES // BM
    return pl.pallas_call(
        _head_body,
        grid=(grid,),
        in_specs=[
            pl.BlockSpec((NC, BM, 16), lambda i: (0, i, 0)),
            pl.BlockSpec((NC, BM, 16), lambda i: (0, i, 0)),
            pl.BlockSpec((BM, DH), lambda i: (i, 0)),
            pl.BlockSpec((BM, DH), lambda i: (i, 0)),
            pl.BlockSpec((D, 16), lambda i: (0, 0)),
            pl.BlockSpec((1, 16), lambda i: (0, 0)),
        ],
        out_specs=pl.BlockSpec((BM, 16), lambda i: (i, 0)),
        out_shape=jax.ShapeDtypeStruct((N_NODES, 16), jnp.float32),
    )(aggp, cnt, hA, hB, wrcatT, bcat)


def kernel(x, edge_index, W1l, b1, W1r, W2l, b2, W2r, W3l, b3, W3r,
           Wal, ba, War, Wsl, bs, Wsr, Wel, be, Wer):
    ei = edge_index.astype(jnp.int32)
    src, dst = ei[0], ei[1]
    pad = E_PAD - N_EDGES
    srcp = jnp.concatenate([src, jnp.zeros((pad,), jnp.int32)])
    dstp = jnp.concatenate([dst, jnp.full((pad,), N_NODES, jnp.int32)])
    src3f = srcp.reshape(NS, NCH_F, CH_F)
    dst3f = dstp.reshape(NS, NCH_F, CH_F)
    src4n = srcp.reshape(NC, NS, NCH_N, CH_N)
    dst4n = dstp.reshape(NC, NS, NCH_N, CH_N)

    zeros_h = jnp.zeros((N_NODES, DH), jnp.float32)
    zeros16 = jnp.zeros((N_NODES, 16), jnp.float32)
    ones16 = jnp.ones((N_NODES, 16), jnp.float32)

    agg_feat = _make_sc_agg_feat()
    agg_n16 = _make_sc_agg_node16()
    dense_relu = _make_dense(True, False)
    dense_relu_proj = _make_dense(True, True)

    cnt = agg_n16(ones16, src4n, dst4n, zeros16)

    xA, xB = x[:, 0:DH], x[:, DH:D]
    a1A, a1B = agg_feat(xA, xB, src3f, dst3f, zeros_h)
    h1A, h1B = dense_relu(a1A, a1B, cnt, xA, xB, W1l.T, W1r.T, b1[None])
    a2A, a2B = agg_feat(h1A, h1B, src3f, dst3f, zeros_h)
    h2A, h2B = dense_relu(a2A, a2B, cnt, h1A, h1B, W2l.T, W2r.T, b2[None])
    a3A, a3B = agg_feat(h2A, h2B, src3f, dst3f, zeros_h)

    WcatT = jnp.pad(jnp.concatenate([Wal, Wsl, Wel], 0), ((0, 8), (0, 0))).T
    WrcatT = jnp.pad(jnp.concatenate([War, Wsr, Wer], 0), ((0, 8), (0, 0))).T
    bcat = jnp.pad(jnp.concatenate([ba, bs, be]), (0, 8))

    h3A, h3B, p16 = dense_relu_proj(a3A, a3B, cnt, h2A, h2B,
                                    W3l.T, W3r.T, b3[None], WcatT)
    aggp = agg_n16(p16, src4n, dst4n, zeros16)
    outc = _head_dense(aggp, cnt, h3A, h3B, WrcatT, bcat[None])
    return outc[:, :3], outc[:, 3:5], outc[:, 5:8]


# speedup vs baseline: 8.1773x; 1.4517x over previous
import functools

import jax
import jax.numpy as jnp
from jax import lax
from jax.experimental import pallas as pl
from jax.experimental.pallas import tpu as pltpu
from jax.experimental.pallas import tpu_sc as plsc

N_NODES = 10000
N_EDGES = 320000
D = 128
DH = D // 2

NC, NS = 2, 16
E_PAD = 327680
CH_F = 256
CH_N = 512
NCH_F = E_PAD // NS // CH_F
NCH_N = E_PAD // (NC * NS) // CH_N
ROWS_A = 632
ROWS_B = N_NODES - (NS - 1) * ROWS_A
N_ACC = N_NODES + 16
N_HALF = N_NODES // NC
NH_ACC = N_HALF + 16
HROWS_A = 312
HROWS_B = N_HALF - (NS - 1) * HROWS_A


def _pipeline(table, src_v, dst_v, rows2, sems, acc_sh, nch, ch):
    sem_g = sems[:2]
    sem_s = sems[2:]

    def g_start(j, b):
        pltpu.async_copy(table.at[src_v.at[j]], rows2.at[b], sem_g[b])

    def g_wait(b):
        pltpu.make_async_copy(table.at[pl.ds(0, ch)], rows2.at[b],
                              sem_g[b]).wait()

    def s_start(j, b):
        pltpu.async_copy(rows2.at[b], acc_sh.at[dst_v.at[j]], sem_s[b],
                         add=True)

    def s_wait(b):
        pltpu.make_async_copy(rows2.at[b], acc_sh.at[pl.ds(0, ch)],
                              sem_s[b]).wait()

    def pair(p, first, last):
        g_wait(0)
        if not first:
            s_wait(1)
        s_start(2 * p, 0)
        g_start(2 * p + 1, 1)
        g_wait(1)
        s_wait(0)
        s_start(2 * p + 1, 1)
        if not last:
            g_start(2 * p + 2, 0)

    np_ = nch // 2

    def steady(p, carry):
        pair(p, False, False)
        return carry

    g_start(0, 0)
    pair(0, True, False)
    lax.fori_loop(1, np_ - 1, steady, 0)
    pair(np_ - 1, False, True)
    s_wait(1)


def _sc_agg_feat_body(*refs):
    (tabA, tabB, src3, dst3, zeros_h, outA, outB, src_v, dst_v,
     rows2, sem_g0, sem_g1, sem_s0, sem_s1, acc_sh) = refs
    c = lax.axis_index("c")
    s = lax.axis_index("s")

    def rows_copy(mk_src, mk_dst):
        @pl.when(s < NS - 1)
        def _():
            pltpu.sync_copy(mk_src(s * ROWS_A, ROWS_A),
                            mk_dst(s * ROWS_A, ROWS_A))

        @pl.when(s == NS - 1)
        def _():
            pltpu.sync_copy(mk_src((NS - 1) * ROWS_A, ROWS_B),
                            mk_dst((NS - 1) * ROWS_A, ROWS_B))

    rows_copy(lambda o, n: zeros_h.at[pl.ds(o, n)],
              lambda o, n: acc_sh.at[pl.ds(o, n)])
    pltpu.sync_copy(src3.at[s], src_v)
    pltpu.sync_copy(dst3.at[s], dst_v)
    plsc.subcore_barrier()

    sems = (sem_g0, sem_g1, sem_s0, sem_s1)

    @pl.when(c == 0)
    def _():
        _pipeline(tabA, src_v, dst_v, rows2, sems, acc_sh, NCH_F, CH_F)

    @pl.when(c == 1)
    def _():
        _pipeline(tabB, src_v, dst_v, rows2, sems, acc_sh, NCH_F, CH_F)

    plsc.subcore_barrier()

    @pl.when(c == 0)
    def _():
        rows_copy(lambda o, n: acc_sh.at[pl.ds(o, n)],
                  lambda o, n: outA.at[pl.ds(o, n)])

    @pl.when(c == 1)
    def _():
        rows_copy(lambda o, n: acc_sh.at[pl.ds(o, n)],
                  lambda o, n: outB.at[pl.ds(o, n)])


def _sc_agg_node16_body(*refs):
    (table, src4, dst4, zeros16, out, src_v, dst_v,
     rows2, sem_g0, sem_g1, sem_s0, sem_s1, acc_sh) = refs
    c = lax.axis_index("c")
    s = lax.axis_index("s")

    def rows_copy(mk_src, mk_dst):
        @pl.when(s < NS - 1)
        def _():
            pltpu.sync_copy(mk_src(s * ROWS_A, ROWS_A),
                            mk_dst(s * ROWS_A, ROWS_A))

        @pl.when(s == NS - 1)
        def _():
            pltpu.sync_copy(mk_src((NS - 1) * ROWS_A, ROWS_B),
                            mk_dst((NS - 1) * ROWS_A, ROWS_B))

    rows_copy(lambda o, n: zeros16.at[pl.ds(o, n)],
              lambda o, n: acc_sh.at[pl.ds(o, n)])
    pltpu.sync_copy(src4.at[c, s], src_v)
    pltpu.sync_copy(dst4.at[c, s], dst_v)
    plsc.subcore_barrier()

    _pipeline(table, src_v, dst_v, rows2,
              (sem_g0, sem_g1, sem_s0, sem_s1), acc_sh, NCH_N, CH_N)
    plsc.subcore_barrier()

    rows_copy(lambda o, n: acc_sh.at[pl.ds(o, n)],
              lambda o, n: out.at[c, pl.ds(o, n)])


@functools.lru_cache(maxsize=None)
def _make_sc_agg_feat():
    mesh = plsc.VectorSubcoreMesh(core_axis_name="c", subcore_axis_name="s",
                                  num_cores=NC, num_subcores=NS)
    return pl.kernel(
        _sc_agg_feat_body,
        out_type=(jax.ShapeDtypeStruct((N_NODES, DH), jnp.bfloat16),
                  jax.ShapeDtypeStruct((N_NODES, DH), jnp.bfloat16)),
        mesh=mesh,
        scratch_types=[
            pltpu.VMEM((NCH_F, CH_F), jnp.int32),
            pltpu.VMEM((NCH_F, CH_F), jnp.int32),
            pltpu.VMEM((2, CH_F, DH), jnp.bfloat16),
            pltpu.SemaphoreType.DMA,
            pltpu.SemaphoreType.DMA,
            pltpu.SemaphoreType.DMA,
            pltpu.SemaphoreType.DMA,
            pltpu.VMEM_SHARED((N_ACC, DH), jnp.bfloat16),
        ],
        compiler_params=pltpu.CompilerParams(use_tc_tiling_on_sc=False),
    )


@functools.lru_cache(maxsize=None)
def _make_sc_agg_node16():
    mesh = plsc.VectorSubcoreMesh(core_axis_name="c", subcore_axis_name="s",
                                  num_cores=NC, num_subcores=NS)
    return pl.kernel(
        _sc_agg_node16_body,
        out_type=jax.ShapeDtypeStruct((NC, N_NODES, 16), jnp.float32),
        mesh=mesh,
        scratch_types=[
            pltpu.VMEM((NCH_N, CH_N), jnp.int32),
            pltpu.VMEM((NCH_N, CH_N), jnp.int32),
            pltpu.VMEM((2, CH_N, 16), jnp.float32),
            pltpu.SemaphoreType.DMA,
            pltpu.SemaphoreType.DMA,
            pltpu.SemaphoreType.DMA,
            pltpu.SemaphoreType.DMA,
            pltpu.VMEM_SHARED((N_ACC, 16), jnp.float32),
        ],
        compiler_params=pltpu.CompilerParams(use_tc_tiling_on_sc=False),
    )


def _dense_body(relu, proj, aggA, aggB, cnt, hA, hB, wl, wr, b, *rest):
    inv = 1.0 / jnp.maximum(cnt[0, :, 0:1] + cnt[1, :, 0:1], 1.0)
    mA = aggA[...].astype(jnp.float32) * inv
    mB = aggB[...].astype(jnp.float32) * inv
    y = (jnp.dot(mA, wl[0:DH], preferred_element_type=jnp.float32)
         + jnp.dot(mB, wl[DH:D], preferred_element_type=jnp.float32)
         + b[...]
         + jnp.dot(hA[...].astype(jnp.float32), wr[0:DH],
                   preferred_element_type=jnp.float32)
         + jnp.dot(hB[...].astype(jnp.float32), wr[DH:D],
                   preferred_element_type=jnp.float32))
    if relu:
        y = jnp.maximum(y, 0.0)
    if proj:
        wp, oA, oB, op16 = rest
        oA[...] = y[:, 0:DH].astype(jnp.bfloat16)
        oB[...] = y[:, DH:D].astype(jnp.bfloat16)
        op16[...] = jnp.dot(y, wp[...], preferred_element_type=jnp.float32)
    else:
        oA, oB = rest
        oA[...] = y[:, 0:DH].astype(jnp.bfloat16)
        oB[...] = y[:, DH:D].astype(jnp.bfloat16)


BM = 1000


def _make_dense(relu, proj):
    grid = N_NODES // BM
    in_specs = [
        pl.BlockSpec((BM, DH), lambda i: (i, 0)),
        pl.BlockSpec((BM, DH), lambda i: (i, 0)),
        pl.BlockSpec((NC, BM, 16), lambda i: (0, i, 0)),
        pl.BlockSpec((BM, DH), lambda i: (i, 0)),
        pl.BlockSpec((BM, DH), lambda i: (i, 0)),
        pl.BlockSpec((D, D), lambda i: (0, 0)),
        pl.BlockSpec((D, D), lambda i: (0, 0)),
        pl.BlockSpec((1, D), lambda i: (0, 0)),
    ]
    out_shape = [jax.ShapeDtypeStruct((N_NODES, DH), jnp.bfloat16),
                 jax.ShapeDtypeStruct((N_NODES, DH), jnp.bfloat16)]
    out_specs = [pl.BlockSpec((BM, DH), lambda i: (i, 0)),
                 pl.BlockSpec((BM, DH), lambda i: (i, 0))]
    if proj:
        in_specs.append(pl.BlockSpec((D, 16), lambda i: (0, 0)))
        out_shape.append(jax.ShapeDtypeStruct((N_NODES, 16), jnp.float32))
        out_specs.append(pl.BlockSpec((BM, 16), lambda i: (i, 0)))

    def wrapped(*args):
        return pl.pallas_call(
            functools.partial(_dense_body, relu, proj),
            grid=(grid,),
            in_specs=in_specs,
            out_specs=out_specs,
            out_shape=out_shape,
        )(*args)

    return wrapped


def _head_body(aggp, cnt, hA, hB, wr, b, out):
    inv = 1.0 / jnp.maximum(cnt[0, :, 0:1] + cnt[1, :, 0:1], 1.0)
    out[...] = ((aggp[0] + aggp[1]) * inv + b[...]
                + jnp.dot(hA[...].astype(jnp.float32), wr[0:DH],
                          preferred_element_type=jnp.float32)
                + jnp.dot(hB[...].astype(jnp.float32), wr[DH:D],
                          preferred_element_type=jnp.float32))


def _head_dense(aggp, cnt, hA, hB, wrcatT, bcat):
    grid = N_NODES // BM
    return pl.pallas_call(
        _head_body,
        grid=(grid,),
        in_specs=[
            pl.BlockSpec((NC, BM, 16), lambda i: (0, i, 0)),
            pl.BlockSpec((NC, BM, 16), lambda i: (0, i, 0)),
            pl.BlockSpec((BM, DH), lambda i: (i, 0)),
            pl.BlockSpec((BM, DH), lambda i: (i, 0)),
            pl.BlockSpec((D, 16), lambda i: (0, 0)),
            pl.BlockSpec((1, 16), lambda i: (0, 0)),
        ],
        out_specs=pl.BlockSpec((BM, 16), lambda i: (i, 0)),
        out_shape=jax.ShapeDtypeStruct((N_NODES, 16), jnp.float32),
    )(aggp, cnt, hA, hB, wrcatT, bcat)


def kernel(x, edge_index, W1l, b1, W1r, W2l, b2, W2r, W3l, b3, W3r,
           Wal, ba, War, Wsl, bs, Wsr, Wel, be, Wer):
    ei = edge_index.astype(jnp.int32)
    src, dst = ei[0], ei[1]
    pad = E_PAD - N_EDGES
    srcp = jnp.concatenate([src, jnp.zeros((pad,), jnp.int32)])
    dstp = jnp.concatenate([dst, jnp.full((pad,), N_NODES, jnp.int32)])
    src3f = srcp.reshape(NS, NCH_F, CH_F)
    dst3f = dstp.reshape(NS, NCH_F, CH_F)
    src4n = srcp.reshape(NC, NS, NCH_N, CH_N)
    dst4n = dstp.reshape(NC, NS, NCH_N, CH_N)

    zeros_h = jnp.zeros((N_NODES, DH), jnp.bfloat16)
    zeros16 = jnp.zeros((N_NODES, 16), jnp.float32)
    ones16 = jnp.ones((N_NODES, 16), jnp.float32)

    agg_feat = _make_sc_agg_feat()
    agg_n16 = _make_sc_agg_node16()
    dense_relu = _make_dense(True, False)
    dense_relu_proj = _make_dense(True, True)

    cnt = agg_n16(ones16, src4n, dst4n, zeros16)

    xA = x[:, 0:DH].astype(jnp.bfloat16)
    xB = x[:, DH:D].astype(jnp.bfloat16)
    a1A, a1B = agg_feat(xA, xB, src3f, dst3f, zeros_h)
    h1A, h1B = dense_relu(a1A, a1B, cnt, xA, xB, W1l.T, W1r.T, b1[None])
    a2A, a2B = agg_feat(h1A, h1B, src3f, dst3f, zeros_h)
    h2A, h2B = dense_relu(a2A, a2B, cnt, h1A, h1B, W2l.T, W2r.T, b2[None])
    a3A, a3B = agg_feat(h2A, h2B, src3f, dst3f, zeros_h)

    WcatT = jnp.pad(jnp.concatenate([Wal, Wsl, Wel], 0), ((0, 8), (0, 0))).T
    WrcatT = jnp.pad(jnp.concatenate([War, Wsr, Wer], 0), ((0, 8), (0, 0))).T
    bcat = jnp.pad(jnp.concatenate([ba, bs, be]), (0, 8))

    h3A, h3B, p16 = dense_relu_proj(a3A, a3B, cnt, h2A, h2B,
                                    W3l.T, W3r.T, b3[None], WcatT)
    aggp = agg_n16(p16, src4n, dst4n, zeros16)
    outc = _head_dense(aggp, cnt, h3A, h3B, WrcatT, bcat[None])
    return outc[:, :3], outc[:, 3:5], outc[:, 5:8]


# speedup vs baseline: 8.3125x; 1.0165x over previous
import functools

import jax
import jax.numpy as jnp
from jax import lax
from jax.experimental import pallas as pl
from jax.experimental.pallas import tpu as pltpu
from jax.experimental.pallas import tpu_sc as plsc

N_NODES = 10000
N_EDGES = 320000
D = 128
DH = D // 2

NC, NS = 2, 16
E_PAD = 327680
CH_F = 256
CH_N = 512
NCH_F = E_PAD // NS // CH_F
NCH_N = E_PAD // (NC * NS) // CH_N
ROWS_A = 632
ROWS_B = N_NODES - (NS - 1) * ROWS_A
N_ACC = N_NODES + 16


def _pipeline(table, src_v, dst_v, rows2, sems, acc_sh, nch, ch):
    sem_g = sems[:2]
    sem_s = sems[2:]

    def g_start(j, b):
        pltpu.async_copy(table.at[src_v.at[j]], rows2.at[b], sem_g[b])

    def g_wait(b):
        pltpu.make_async_copy(table.at[pl.ds(0, ch)], rows2.at[b],
                              sem_g[b]).wait()

    def s_start(j, b):
        pltpu.async_copy(rows2.at[b], acc_sh.at[dst_v.at[j]], sem_s[b],
                         add=True)

    def s_wait(b):
        pltpu.make_async_copy(rows2.at[b], acc_sh.at[pl.ds(0, ch)],
                              sem_s[b]).wait()

    def pair(p, first, last):
        g_wait(0)
        if not first:
            s_wait(1)
        s_start(2 * p, 0)
        g_start(2 * p + 1, 1)
        g_wait(1)
        s_wait(0)
        s_start(2 * p + 1, 1)
        if not last:
            g_start(2 * p + 2, 0)

    np_ = nch // 2

    def steady(p, carry):
        pair(p, False, False)
        return carry

    g_start(0, 0)
    pair(0, True, False)
    lax.fori_loop(1, np_ - 1, steady, 0)
    pair(np_ - 1, False, True)
    s_wait(1)


def _sc_agg_feat_body(*refs):
    (tabA, tabB, src3, dst3, zeros_h, outA, outB, src_v, dst_v,
     rows2, sem_g0, sem_g1, sem_s0, sem_s1, acc_sh) = refs
    c = lax.axis_index("c")
    s = lax.axis_index("s")

    def rows_copy(mk_src, mk_dst):
        @pl.when(s < NS - 1)
        def _():
            pltpu.sync_copy(mk_src(s * ROWS_A, ROWS_A),
                            mk_dst(s * ROWS_A, ROWS_A))

        @pl.when(s == NS - 1)
        def _():
            pltpu.sync_copy(mk_src((NS - 1) * ROWS_A, ROWS_B),
                            mk_dst((NS - 1) * ROWS_A, ROWS_B))

    rows_copy(lambda o, n: zeros_h.at[pl.ds(o, n)],
              lambda o, n: acc_sh.at[pl.ds(o, n)])
    pltpu.sync_copy(src3.at[s], src_v)
    pltpu.sync_copy(dst3.at[s], dst_v)
    plsc.subcore_barrier()

    sems = (sem_g0, sem_g1, sem_s0, sem_s1)

    @pl.when(c == 0)
    def _():
        _pipeline(tabA, src_v, dst_v, rows2, sems, acc_sh, NCH_F, CH_F)

    @pl.when(c == 1)
    def _():
        _pipeline(tabB, src_v, dst_v, rows2, sems, acc_sh, NCH_F, CH_F)

    plsc.subcore_barrier()

    @pl.when(c == 0)
    def _():
        rows_copy(lambda o, n: acc_sh.at[pl.ds(o, n)],
                  lambda o, n: outA.at[pl.ds(o, n)])

    @pl.when(c == 1)
    def _():
        rows_copy(lambda o, n: acc_sh.at[pl.ds(o, n)],
                  lambda o, n: outB.at[pl.ds(o, n)])


def _sc_agg_node16_body(*refs):
    (table, src4, dst4, zeros16, out, src_v, dst_v,
     rows2, sem_g0, sem_g1, sem_s0, sem_s1, acc_sh) = refs
    c = lax.axis_index("c")
    s = lax.axis_index("s")

    def rows_copy(mk_src, mk_dst):
        @pl.when(s < NS - 1)
        def _():
            pltpu.sync_copy(mk_src(s * ROWS_A, ROWS_A),
                            mk_dst(s * ROWS_A, ROWS_A))

        @pl.when(s == NS - 1)
        def _():
            pltpu.sync_copy(mk_src((NS - 1) * ROWS_A, ROWS_B),
                            mk_dst((NS - 1) * ROWS_A, ROWS_B))

    rows_copy(lambda o, n: zeros16.at[pl.ds(o, n)],
              lambda o, n: acc_sh.at[pl.ds(o, n)])
    pltpu.sync_copy(src4.at[c, s], src_v)
    pltpu.sync_copy(dst4.at[c, s], dst_v)
    plsc.subcore_barrier()

    _pipeline(table, src_v, dst_v, rows2,
              (sem_g0, sem_g1, sem_s0, sem_s1), acc_sh, NCH_N, CH_N)
    plsc.subcore_barrier()

    rows_copy(lambda o, n: acc_sh.at[pl.ds(o, n)],
              lambda o, n: out.at[c, pl.ds(o, n)])


@functools.lru_cache(maxsize=None)
def _make_sc_agg_feat():
    mesh = plsc.VectorSubcoreMesh(core_axis_name="c", subcore_axis_name="s",
                                  num_cores=NC, num_subcores=NS)
    return pl.kernel(
        _sc_agg_feat_body,
        out_type=(jax.ShapeDtypeStruct((N_NODES, DH), jnp.bfloat16),
                  jax.ShapeDtypeStruct((N_NODES, DH), jnp.bfloat16)),
        mesh=mesh,
        scratch_types=[
            pltpu.VMEM((NCH_F, CH_F), jnp.int32),
            pltpu.VMEM((NCH_F, CH_F), jnp.int32),
            pltpu.VMEM((2, CH_F, DH), jnp.bfloat16),
            pltpu.SemaphoreType.DMA,
            pltpu.SemaphoreType.DMA,
            pltpu.SemaphoreType.DMA,
            pltpu.SemaphoreType.DMA,
            pltpu.VMEM_SHARED((N_ACC, DH), jnp.bfloat16),
        ],
        compiler_params=pltpu.CompilerParams(use_tc_tiling_on_sc=False),
    )


@functools.lru_cache(maxsize=None)
def _make_sc_agg_node16():
    mesh = plsc.VectorSubcoreMesh(core_axis_name="c", subcore_axis_name="s",
                                  num_cores=NC, num_subcores=NS)
    return pl.kernel(
        _sc_agg_node16_body,
        out_type=jax.ShapeDtypeStruct((NC, N_NODES, 16), jnp.float32),
        mesh=mesh,
        scratch_types=[
            pltpu.VMEM((NCH_N, CH_N), jnp.int32),
            pltpu.VMEM((NCH_N, CH_N), jnp.int32),
            pltpu.VMEM((2, CH_N, 16), jnp.float32),
            pltpu.SemaphoreType.DMA,
            pltpu.SemaphoreType.DMA,
            pltpu.SemaphoreType.DMA,
            pltpu.SemaphoreType.DMA,
            pltpu.VMEM_SHARED((N_ACC, 16), jnp.float32),
        ],
        compiler_params=pltpu.CompilerParams(use_tc_tiling_on_sc=False),
    )


def _dense_body(relu, proj, aggA, aggB, cnt, hA, hB, wl, wr, b, *rest):
    inv = 1.0 / jnp.maximum(cnt[0, :, 0:1] + cnt[1, :, 0:1], 1.0)
    mA = aggA[...].astype(jnp.float32) * inv
    mB = aggB[...].astype(jnp.float32) * inv
    y = (jnp.dot(mA, wl[0:DH], preferred_element_type=jnp.float32)
         + jnp.dot(mB, wl[DH:D], preferred_element_type=jnp.float32)
         + b[...]
         + jnp.dot(hA[...].astype(jnp.float32), wr[0:DH],
                   preferred_element_type=jnp.float32)
         + jnp.dot(hB[...].astype(jnp.float32), wr[DH:D],
                   preferred_element_type=jnp.float32))
    if relu:
        y = jnp.maximum(y, 0.0)
    if proj:
        wp, oA, oB, op16 = rest
        oA[...] = y[:, 0:DH].astype(jnp.bfloat16)
        oB[...] = y[:, DH:D].astype(jnp.bfloat16)
        op16[...] = jnp.dot(y, wp[...], preferred_element_type=jnp.float32)
    else:
        oA, oB = rest
        oA[...] = y[:, 0:DH].astype(jnp.bfloat16)
        oB[...] = y[:, DH:D].astype(jnp.bfloat16)


BM = 2000


def _make_dense(relu, proj):
    grid = N_NODES // BM
    in_specs = [
        pl.BlockSpec((BM, DH), lambda i: (i, 0)),
        pl.BlockSpec((BM, DH), lambda i: (i, 0)),
        pl.BlockSpec((NC, BM, 16), lambda i: (0, i, 0)),
        pl.BlockSpec((BM, DH), lambda i: (i, 0)),
        pl.BlockSpec((BM, DH), lambda i: (i, 0)),
        pl.BlockSpec((D, D), lambda i: (0, 0)),
        pl.BlockSpec((D, D), lambda i: (0, 0)),
        pl.BlockSpec((1, D), lambda i: (0, 0)),
    ]
    out_shape = [jax.ShapeDtypeStruct((N_NODES, DH), jnp.bfloat16),
                 jax.ShapeDtypeStruct((N_NODES, DH), jnp.bfloat16)]
    out_specs = [pl.BlockSpec((BM, DH), lambda i: (i, 0)),
                 pl.BlockSpec((BM, DH), lambda i: (i, 0))]
    if proj:
        in_specs.append(pl.BlockSpec((D, 16), lambda i: (0, 0)))
        out_shape.append(jax.ShapeDtypeStruct((N_NODES, 16), jnp.float32))
        out_specs.append(pl.BlockSpec((BM, 16), lambda i: (i, 0)))

    def wrapped(*args):
        return pl.pallas_call(
            functools.partial(_dense_body, relu, proj),
            grid=(grid,),
            in_specs=in_specs,
            out_specs=out_specs,
            out_shape=out_shape,
        )(*args)

    return wrapped


def _head_body(aggp, cnt, hA, hB, wr, b, out):
    inv = 1.0 / jnp.maximum(cnt[0, :, 0:1] + cnt[1, :, 0:1], 1.0)
    out[...] = ((aggp[0] + aggp[1]) * inv + b[...]
                + jnp.dot(hA[...].astype(jnp.float32), wr[0:DH],
                          preferred_element_type=jnp.float32)
                + jnp.dot(hB[...].astype(jnp.float32), wr[DH:D],
                          preferred_element_type=jnp.float32))


def _head_dense(aggp, cnt, hA, hB, wrcatT, bcat):
    grid = N_NODES // BM
    return pl.pallas_call(
        _head_body,
        grid=(grid,),
        in_specs=[
            pl.BlockSpec((NC, BM, 16), lambda i: (0, i, 0)),
            pl.BlockSpec((NC, BM, 16), lambda i: (0, i, 0)),
            pl.BlockSpec((BM, DH), lambda i: (i, 0)),
            pl.BlockSpec((BM, DH), lambda i: (i, 0)),
            pl.BlockSpec((D, 16), lambda i: (0, 0)),
            pl.BlockSpec((1, 16), lambda i: (0, 0)),
        ],
        out_specs=pl.BlockSpec((BM, 16), lambda i: (i, 0)),
        out_shape=jax.ShapeDtypeStruct((N_NODES, 16), jnp.float32),
    )(aggp, cnt, hA, hB, wrcatT, bcat)


def kernel(x, edge_index, W1l, b1, W1r, W2l, b2, W2r, W3l, b3, W3r,
           Wal, ba, War, Wsl, bs, Wsr, Wel, be, Wer):
    ei = edge_index.astype(jnp.int32)
    src, dst = ei[0], ei[1]
    pad = E_PAD - N_EDGES
    srcp = jnp.concatenate([src, jnp.zeros((pad,), jnp.int32)])
    dstp = jnp.concatenate([dst, jnp.full((pad,), N_NODES, jnp.int32)])
    src3f = srcp.reshape(NS, NCH_F, CH_F)
    dst3f = dstp.reshape(NS, NCH_F, CH_F)
    src4n = srcp.reshape(NC, NS, NCH_N, CH_N)
    dst4n = dstp.reshape(NC, NS, NCH_N, CH_N)

    zeros_h = jnp.zeros((N_NODES, DH), jnp.bfloat16)
    zeros16 = jnp.zeros((N_NODES, 16), jnp.float32)
    ones16 = jnp.ones((N_NODES, 16), jnp.float32)

    agg_feat = _make_sc_agg_feat()
    agg_n16 = _make_sc_agg_node16()
    dense_relu = _make_dense(True, False)
    dense_relu_proj = _make_dense(True, True)

    cnt = agg_n16(ones16, src4n, dst4n, zeros16)

    xA = x[:, 0:DH].astype(jnp.bfloat16)
    xB = x[:, DH:D].astype(jnp.bfloat16)
    a1A, a1B = agg_feat(xA, xB, src3f, dst3f, zeros_h)
    h1A, h1B = dense_relu(a1A, a1B, cnt, xA, xB, W1l.T, W1r.T, b1[None])
    a2A, a2B = agg_feat(h1A, h1B, src3f, dst3f, zeros_h)
    h2A, h2B = dense_relu(a2A, a2B, cnt, h1A, h1B, W2l.T, W2r.T, b2[None])
    a3A, a3B = agg_feat(h2A, h2B, src3f, dst3f, zeros_h)

    WcatT = jnp.pad(jnp.concatenate([Wal, Wsl, Wel], 0), ((0, 8), (0, 0))).T
    WrcatT = jnp.pad(jnp.concatenate([War, Wsr, Wer], 0), ((0, 8), (0, 0))).T
    bcat = jnp.pad(jnp.concatenate([ba, bs, be]), (0, 8))

    h3A, h3B, p16 = dense_relu_proj(a3A, a3B, cnt, h2A, h2B,
                                    W3l.T, W3r.T, b3[None], WcatT)
    aggp = agg_n16(p16, src4n, dst4n, zeros16)
    outc = _head_dense(aggp, cnt, h3A, h3B, WrcatT, bcat[None])
    return outc[:, :3], outc[:, 3:5], outc[:, 5:8]
